# baseline (speedup 1.0000x reference)
import functools

import jax
import jax.numpy as jnp
from jax import lax
from jax.experimental import pallas as pl
from jax.experimental.pallas import tpu as pltpu
from jax.experimental.pallas import tpu_sc as plsc

D = 256
K = 8192
N = 8192
RT = 256
KC = 1024
NKC = K // KC
CT = 1024
BETA = 0.25


def _k1_body(x_ref, e_ref, idx_ref, loss_ref,
             e2_ref, x2_ref, a_ref, g_ref, acc_ref):
    i = pl.program_id(0)
    j = pl.program_id(1)

    @pl.when(jnp.logical_and(i == 0, j == 0))
    def _():
        acc_ref[0, 0] = 0.0

    eb = e_ref[:, pl.ds(j * KC, KC)]

    @pl.when(i == 0)
    def _():
        e2_ref[:, pl.ds(j * KC, KC)] = jnp.sum(eb * eb, axis=0, keepdims=True)

    @pl.when(j == 0)
    def _():
        xb0 = x_ref[...]
        x2_ref[...] = jnp.sum(xb0 * xb0, axis=1, keepdims=True)

    xb = x_ref[...]
    m = lax.dot_general(xb, eb, (((1,), (0,)), ((), ())),
                        preferred_element_type=jnp.float32)
    d = (x2_ref[...] - 2.0 * m) + e2_ref[:, pl.ds(j * KC, KC)]

    @pl.when(j == 0)
    def _():
        a_ref[...] = d
        g_ref[...] = jnp.zeros((RT, KC), jnp.int32)

    @pl.when(j != 0)
    def _():
        a = a_ref[...]
        p = d < a
        a_ref[...] = jnp.where(p, d, a)
        g_ref[...] = jnp.where(p, j, g_ref[...])

    @pl.when(j == NKC - 1)
    def _():
        a = a_ref[...]
        dmin = jnp.min(a, axis=1, keepdims=True)
        acc_ref[0, 0] += jnp.sum(dmin)
        ids = lax.broadcasted_iota(jnp.int32, (RT, KC), 1)
        cand = jnp.where(a == dmin, g_ref[...] * KC + ids,
                         jnp.int32(2147483647))
        idx_ref[...] = jnp.min(cand, axis=1, keepdims=True)

    @pl.when(jnp.logical_and(i == pl.num_programs(0) - 1, j == NKC - 1))
    def _():
        loss_ref[...] = jnp.full((1, 1), (1.0 + BETA) / (N * D),
                                 jnp.float32) * acc_ref[0, 0]


def _k1(xf, emb):
    return pl.pallas_call(
        _k1_body,
        grid=(N // RT, NKC),
        in_specs=[pl.BlockSpec((RT, D), lambda i, j: (i, 0)),
                  pl.BlockSpec((D, K), lambda i, j: (0, 0))],
        out_specs=[pl.BlockSpec((RT, 1), lambda i, j: (i, 0)),
                   pl.BlockSpec((1, 1), lambda i, j: (0, 0))],
        out_shape=[jax.ShapeDtypeStruct((N, 1), jnp.int32),
                   jax.ShapeDtypeStruct((1, 1), jnp.float32)],
        scratch_shapes=[pltpu.VMEM((1, K), jnp.float32),
                        pltpu.VMEM((RT, 1), jnp.float32),
                        pltpu.VMEM((RT, KC), jnp.float32),
                        pltpu.VMEM((RT, KC), jnp.int32),
                        pltpu.SMEM((1, 1), jnp.float32)],
    )(xf, emb)


def _k3_body(idx_ref, out_ref, cnt_ref):
    i = pl.program_id(0)

    @pl.when(i == 0)
    def _():
        cnt_ref[...] = jnp.zeros_like(cnt_ref)

    idxb = idx_ref[...]
    ids = lax.broadcasted_iota(jnp.int32, (CT, K), 1)
    eq = (idxb == ids).astype(jnp.float32)
    cnt_ref[...] += jnp.sum(eq, axis=0, keepdims=True)

    @pl.when(i == pl.num_programs(0) - 1)
    def _():
        p = cnt_ref[...] * (1.0 / N)
        h = -jnp.sum(p * jnp.log(p + 1e-10))
        out_ref[...] = jnp.exp(h) * jnp.ones((1, 1), jnp.float32)


def _k3(idx2):
    return pl.pallas_call(
        _k3_body,
        grid=(N // CT,),
        in_specs=[pl.BlockSpec((CT, 1), lambda i: (i, 0))],
        out_specs=pl.BlockSpec((1, 1), lambda i: (0, 0)),
        out_shape=jax.ShapeDtypeStruct((1, 1), jnp.float32),
        scratch_shapes=[pltpu.VMEM((1, K), jnp.float32)],
    )(idx2)


_NW = 32
_BPW = N // _NW


@functools.cache
def _k2_build():
    @functools.partial(
        pl.kernel,
        mesh=plsc.VectorSubcoreMesh(core_axis_name="c", subcore_axis_name="s"),
        out_type=jax.ShapeDtypeStruct((N, D), jnp.float32),
        scratch_types=[pltpu.VMEM((_BPW,), jnp.int32),
                       pltpu.VMEM((_BPW, D), jnp.float32),
                       pltpu.SemaphoreType.DMA],
    )
    def _k2(table_hbm, idx_hbm, out_hbm, idx_v, rows_v, sem):
        wid = lax.axis_index("s") * 2 + lax.axis_index("c")
        base = wid * _BPW
        pltpu.sync_copy(idx_hbm.at[pl.ds(base, _BPW)], idx_v)
        pltpu.async_copy(table_hbm.at[idx_v], rows_v, sem).wait()
        pltpu.sync_copy(rows_v, out_hbm.at[pl.ds(base, _BPW)])

    return _k2


def kernel(x, embeddings):
    xf = x.reshape(N, D)
    idx2, loss = _k1(xf, embeddings)
    quant = _k2_build()(embeddings.T, idx2.reshape(N))
    perp = _k3(idx2)
    return (quant.reshape(x.shape), loss.reshape(()), perp.reshape(()),
            idx2.reshape(x.shape[:-1]))


# speedup vs baseline: 2.0343x; 2.0343x over previous
import functools

import jax
import jax.numpy as jnp
from jax import lax
from jax.experimental import pallas as pl
from jax.experimental.pallas import tpu as pltpu
from jax.experimental.pallas import tpu_sc as plsc

D = 256
K = 8192
N = 8192
RT = 256
BETA = 0.25


def _k1_body(x_ref, e_ref, idx_ref, loss_ref, perp_ref,
             es_ref, e2_ref, iotaf_ref, cnt_ref, acc_ref):
    i = pl.program_id(0)

    @pl.when(i == 0)
    def _():
        e = e_ref[...]
        e2_ref[...] = jnp.sum(e * e, axis=0, keepdims=True)
        es_ref[...] = -2.0 * e
        iotaf_ref[...] = lax.broadcasted_iota(jnp.int32, (1, K),
                                              1).astype(jnp.float32)
        acc_ref[0, 0] = 0.0
        cnt_ref[...] = jnp.zeros_like(cnt_ref)

    xb = x_ref[...]
    x2 = jnp.sum(xb * xb, axis=1, keepdims=True)
    m2 = lax.dot_general(xb, es_ref[...], (((1,), (0,)), ((), ())),
                         preferred_element_type=jnp.float32)
    d = (x2 + m2) + e2_ref[...]
    dmin = jnp.min(d, axis=1, keepdims=True)
    acc_ref[0, 0] += jnp.sum(dmin)
    cand = jnp.where(d == dmin, iotaf_ref[...], jnp.float32(3.0e38))
    idxf = jnp.min(cand, axis=1, keepdims=True)
    idx = idxf.astype(jnp.int32)
    idx_ref[...] = idx
    hi = idx >> 7
    lo = idx & 127
    ih = lax.broadcasted_iota(jnp.int32, (RT, 64), 1)
    il = lax.broadcasted_iota(jnp.int32, (RT, 128), 1)
    ohh = (hi == ih).astype(jnp.float32)
    ohl = (lo == il).astype(jnp.float32)
    cnt_ref[...] += lax.dot_general(ohh, ohl, (((0,), (0,)), ((), ())),
                                    preferred_element_type=jnp.float32)

    @pl.when(i == pl.num_programs(0) - 1)
    def _():
        loss_ref[...] = jnp.full((1, 1), (1.0 + BETA) / (N * D),
                                 jnp.float32) * acc_ref[0, 0]
        p = cnt_ref[...] * (1.0 / N)
        h = -jnp.sum(p * jnp.log(p + 1e-10))
        perp_ref[...] = jnp.exp(h) * jnp.ones((1, 1), jnp.float32)


def _k1(xf, emb):
    return pl.pallas_call(
        _k1_body,
        grid=(N // RT,),
        in_specs=[pl.BlockSpec((RT, D), lambda i: (i, 0)),
                  pl.BlockSpec((D, K), lambda i: (0, 0))],
        out_specs=[pl.BlockSpec((RT, 1), lambda i: (i, 0)),
                   pl.BlockSpec((1, 1), lambda i: (0, 0)),
                   pl.BlockSpec((1, 1), lambda i: (0, 0))],
        out_shape=[jax.ShapeDtypeStruct((N, 1), jnp.int32),
                   jax.ShapeDtypeStruct((1, 1), jnp.float32),
                   jax.ShapeDtypeStruct((1, 1), jnp.float32)],
        scratch_shapes=[pltpu.VMEM((D, K), jnp.float32),
                        pltpu.VMEM((1, K), jnp.float32),
                        pltpu.VMEM((1, K), jnp.float32),
                        pltpu.VMEM((64, 128), jnp.float32),
                        pltpu.SMEM((1, 1), jnp.float32)],
    )(xf, emb)


_NW = 32
_BPW = N // _NW


@functools.cache
def _k2_build():
    @functools.partial(
        pl.kernel,
        mesh=plsc.VectorSubcoreMesh(core_axis_name="c", subcore_axis_name="s"),
        out_type=jax.ShapeDtypeStruct((N, D), jnp.float32),
        scratch_types=[pltpu.VMEM((_BPW,), jnp.int32),
                       pltpu.VMEM((_BPW, D), jnp.float32),
                       pltpu.SemaphoreType.DMA],
    )
    def _k2(table_hbm, idx_hbm, out_hbm, idx_v, rows_v, sem):
        wid = lax.axis_index("s") * 2 + lax.axis_index("c")
        base = wid * _BPW
        pltpu.sync_copy(idx_hbm.at[pl.ds(base, _BPW)], idx_v)
        pltpu.async_copy(table_hbm.at[idx_v], rows_v, sem).wait()
        pltpu.sync_copy(rows_v, out_hbm.at[pl.ds(base, _BPW)])

    return _k2


def kernel(x, embeddings):
    xf = x.reshape(N, D)
    idx2, loss, perp = _k1(xf, embeddings)
    quant = _k2_build()(embeddings.T, idx2.reshape(N))
    return (quant.reshape(x.shape), loss.reshape(()), perp.reshape(()),
            idx2.reshape(x.shape[:-1]))


# speedup vs baseline: 2.1752x; 1.0692x over previous
import functools

import jax
import jax.numpy as jnp
from jax import lax
from jax.experimental import pallas as pl
from jax.experimental.pallas import tpu as pltpu
from jax.experimental.pallas import tpu_sc as plsc

D = 256
K = 8192
N = 8192
RT = 512
BETA = 0.25


def _k1_body(x_ref, e_ref, idx_ref, loss_ref, perp_ref,
             es_ref, e2_ref, iotaf_ref, cnt_ref, acc_ref):
    i = pl.program_id(0)

    @pl.when(i == 0)
    def _():
        e = e_ref[...]
        e2_ref[...] = jnp.sum(e * e, axis=0, keepdims=True)
        es_ref[...] = -2.0 * e
        iotaf_ref[...] = lax.broadcasted_iota(jnp.int32, (1, K),
                                              1).astype(jnp.float32)
        acc_ref[0, 0] = 0.0
        cnt_ref[...] = jnp.zeros_like(cnt_ref)

    xb = x_ref[...]
    x2 = jnp.sum(xb * xb, axis=1, keepdims=True)
    m2 = lax.dot_general(xb, es_ref[...], (((1,), (0,)), ((), ())),
                         preferred_element_type=jnp.float32)
    d = (x2 + m2) + e2_ref[...]
    dmin = jnp.min(d, axis=1, keepdims=True)
    acc_ref[0, 0] += jnp.sum(dmin)
    cand = jnp.where(d == dmin, iotaf_ref[...], jnp.float32(3.0e38))
    idxf = jnp.min(cand, axis=1, keepdims=True)
    idx = idxf.astype(jnp.int32)
    idx_ref[...] = idx
    hi = idx >> 7
    lo = idx & 127
    ih = lax.broadcasted_iota(jnp.int32, (RT, 64), 1)
    il = lax.broadcasted_iota(jnp.int32, (RT, 128), 1)
    ohh = (hi == ih).astype(jnp.float32)
    ohl = (lo == il).astype(jnp.float32)
    cnt_ref[...] += lax.dot_general(ohh, ohl, (((0,), (0,)), ((), ())),
                                    preferred_element_type=jnp.float32)

    @pl.when(i == pl.num_programs(0) - 1)
    def _():
        loss_ref[...] = jnp.full((1, 1), (1.0 + BETA) / (N * D),
                                 jnp.float32) * acc_ref[0, 0]
        p = cnt_ref[...] * (1.0 / N)
        h = -jnp.sum(p * jnp.log(p + 1e-10))
        perp_ref[...] = jnp.exp(h) * jnp.ones((1, 1), jnp.float32)


def _k1(xf, emb):
    return pl.pallas_call(
        _k1_body,
        grid=(N // RT,),
        in_specs=[pl.BlockSpec((RT, D), lambda i: (i, 0)),
                  pl.BlockSpec((D, K), lambda i: (0, 0))],
        out_specs=[pl.BlockSpec((RT, 1), lambda i: (i, 0)),
                   pl.BlockSpec((1, 1), lambda i: (0, 0)),
                   pl.BlockSpec((1, 1), lambda i: (0, 0))],
        out_shape=[jax.ShapeDtypeStruct((N, 1), jnp.int32),
                   jax.ShapeDtypeStruct((1, 1), jnp.float32),
                   jax.ShapeDtypeStruct((1, 1), jnp.float32)],
        scratch_shapes=[pltpu.VMEM((D, K), jnp.float32),
                        pltpu.VMEM((1, K), jnp.float32),
                        pltpu.VMEM((1, K), jnp.float32),
                        pltpu.VMEM((64, 128), jnp.float32),
                        pltpu.SMEM((1, 1), jnp.float32)],
    )(xf, emb)


_NW = 32
_BPW = N // _NW


@functools.cache
def _k2_build():
    @functools.partial(
        pl.kernel,
        mesh=plsc.VectorSubcoreMesh(core_axis_name="c", subcore_axis_name="s"),
        out_type=jax.ShapeDtypeStruct((N, D), jnp.float32),
        scratch_types=[pltpu.VMEM((_BPW,), jnp.int32),
                       pltpu.VMEM((_BPW, D), jnp.float32),
                       pltpu.SemaphoreType.DMA],
    )
    def _k2(table_hbm, idx_hbm, out_hbm, idx_v, rows_v, sem):
        wid = lax.axis_index("s") * 2 + lax.axis_index("c")
        base = wid * _BPW
        pltpu.sync_copy(idx_hbm.at[pl.ds(base, _BPW)], idx_v)
        pltpu.async_copy(table_hbm.at[idx_v], rows_v, sem).wait()
        pltpu.sync_copy(rows_v, out_hbm.at[pl.ds(base, _BPW)])

    return _k2


def kernel(x, embeddings):
    xf = x.reshape(N, D)
    idx2, loss, perp = _k1(xf, embeddings)
    quant = _k2_build()(embeddings.T, idx2.reshape(N))
    return (quant.reshape(x.shape), loss.reshape(()), perp.reshape(()),
            idx2.reshape(x.shape[:-1]))


# speedup vs baseline: 2.2554x; 1.0368x over previous
import functools

import jax
import jax.numpy as jnp
from jax import lax
from jax.experimental import pallas as pl
from jax.experimental.pallas import tpu as pltpu
from jax.experimental.pallas import tpu_sc as plsc

D = 256
K = 8192
N = 8192
RT = 1024
BETA = 0.25


def _k1_body(x_ref, e_ref, idx_ref, loss_ref, perp_ref,
             es_ref, e2_ref, iotaf_ref, cnt_ref, acc_ref):
    i = pl.program_id(0)

    @pl.when(i == 0)
    def _():
        e = e_ref[...]
        e2_ref[...] = jnp.sum(e * e, axis=0, keepdims=True)
        es_ref[...] = -2.0 * e
        iotaf_ref[...] = lax.broadcasted_iota(jnp.int32, (1, K),
                                              1).astype(jnp.float32)
        acc_ref[0, 0] = 0.0
        cnt_ref[...] = jnp.zeros_like(cnt_ref)

    xb = x_ref[...]
    x2 = jnp.sum(xb * xb, axis=1, keepdims=True)
    m2 = lax.dot_general(xb, es_ref[...], (((1,), (0,)), ((), ())),
                         preferred_element_type=jnp.float32)
    d = (x2 + m2) + e2_ref[...]
    dmin = jnp.min(d, axis=1, keepdims=True)
    acc_ref[0, 0] += jnp.sum(dmin)
    cand = jnp.where(d == dmin, iotaf_ref[...], jnp.float32(3.0e38))
    idxf = jnp.min(cand, axis=1, keepdims=True)
    idx = idxf.astype(jnp.int32)
    idx_ref[...] = idx
    hi = idx >> 7
    lo = idx & 127
    ih = lax.broadcasted_iota(jnp.int32, (RT, 64), 1)
    il = lax.broadcasted_iota(jnp.int32, (RT, 128), 1)
    ohh = (hi == ih).astype(jnp.float32)
    ohl = (lo == il).astype(jnp.float32)
    cnt_ref[...] += lax.dot_general(ohh, ohl, (((0,), (0,)), ((), ())),
                                    preferred_element_type=jnp.float32)

    @pl.when(i == pl.num_programs(0) - 1)
    def _():
        loss_ref[...] = jnp.full((1, 1), (1.0 + BETA) / (N * D),
                                 jnp.float32) * acc_ref[0, 0]
        p = cnt_ref[...] * (1.0 / N)
        h = -jnp.sum(p * jnp.log(p + 1e-10))
        perp_ref[...] = jnp.exp(h) * jnp.ones((1, 1), jnp.float32)


def _k1(xf, emb):
    return pl.pallas_call(
        _k1_body,
        grid=(N // RT,),
        in_specs=[pl.BlockSpec((RT, D), lambda i: (i, 0)),
                  pl.BlockSpec((D, K), lambda i: (0, 0))],
        out_specs=[pl.BlockSpec((RT, 1), lambda i: (i, 0)),
                   pl.BlockSpec((1, 1), lambda i: (0, 0)),
                   pl.BlockSpec((1, 1), lambda i: (0, 0))],
        out_shape=[jax.ShapeDtypeStruct((N, 1), jnp.int32),
                   jax.ShapeDtypeStruct((1, 1), jnp.float32),
                   jax.ShapeDtypeStruct((1, 1), jnp.float32)],
        scratch_shapes=[pltpu.VMEM((D, K), jnp.float32),
                        pltpu.VMEM((1, K), jnp.float32),
                        pltpu.VMEM((1, K), jnp.float32),
                        pltpu.VMEM((64, 128), jnp.float32),
                        pltpu.SMEM((1, 1), jnp.float32)],
    )(xf, emb)


_NW = 32
_BPW = N // _NW


@functools.cache
def _k2_build():
    @functools.partial(
        pl.kernel,
        mesh=plsc.VectorSubcoreMesh(core_axis_name="c", subcore_axis_name="s"),
        out_type=jax.ShapeDtypeStruct((N, D), jnp.float32),
        scratch_types=[pltpu.VMEM((_BPW,), jnp.int32),
                       pltpu.VMEM((_BPW, D), jnp.float32),
                       pltpu.SemaphoreType.DMA],
    )
    def _k2(table_hbm, idx_hbm, out_hbm, idx_v, rows_v, sem):
        wid = lax.axis_index("s") * 2 + lax.axis_index("c")
        base = wid * _BPW
        pltpu.sync_copy(idx_hbm.at[pl.ds(base, _BPW)], idx_v)
        pltpu.async_copy(table_hbm.at[idx_v], rows_v, sem).wait()
        pltpu.sync_copy(rows_v, out_hbm.at[pl.ds(base, _BPW)])

    return _k2


def kernel(x, embeddings):
    xf = x.reshape(N, D)
    idx2, loss, perp = _k1(xf, embeddings)
    quant = _k2_build()(embeddings.T, idx2.reshape(N))
    return (quant.reshape(x.shape), loss.reshape(()), perp.reshape(()),
            idx2.reshape(x.shape[:-1]))
